# Initial kernel scaffold; baseline (speedup 1.0000x reference)
import jax
import jax.numpy as jnp
from jax import lax
from jax.experimental import pallas as pl
from jax.experimental.pallas import tpu as pltpu
from jax.experimental.pallas import tpu_sc as plsc

N1 = 50000
NG = 50000
NTOT = N1 + NG
E1 = 160000
E2 = 160000
EW = E1 + E2
ETOT = 2 * EW
D = 64
PI = 3.14159265
LN2 = 0.6931471805599453

NTILES = 32
EPT = ETOT // NTILES
STG = 2000
NSTG = EPT // STG
CHUNK = 16384
NCHUNK = 7
NPAD = NCHUNK * CHUNK
K = 128
BROWS = (EPT + NCHUNK * (K - 1) + K - 1) // K
ROWS_PER_TILE = CHUNK // 16


def _ss(h):
    return jnp.log(jnp.exp(h) + 1.0) - LN2


def _lin1_body(xin_ref, w_ref, b_ref, o_ref):
    o_ref[...] = (
        jnp.dot(xin_ref[...], w_ref[...], preferred_element_type=jnp.float32)
        + b_ref[...]
    )


def _lin1(xin, W1, b1):
    blk = 1000
    grid = NTOT // blk
    return pl.pallas_call(
        _lin1_body,
        grid=(grid,),
        in_specs=[
            pl.BlockSpec((blk, D), lambda i: (i, 0)),
            pl.BlockSpec((D, D), lambda i: (0, 0)),
            pl.BlockSpec((1, D), lambda i: (0, 0)),
        ],
        out_specs=pl.BlockSpec((blk, D), lambda i: (i, 0)),
        out_shape=jax.ShapeDtypeStruct((NTOT, D), jnp.float32),
    )(xin, W1, b1.reshape(1, D))


def _filt1_body(rbf_ref, dist_ref, cut_ref, w1_ref, b1_ref, w2_ref, b2_ref, o_ref):
    h = jnp.dot(rbf_ref[...], w1_ref[...], preferred_element_type=jnp.float32)
    h = _ss(h + b1_ref[...])
    h = jnp.dot(h, w2_ref[...], preferred_element_type=jnp.float32)
    h = _ss(h + b2_ref[...])
    env = 1.0 + jnp.cos(PI * dist_ref[...] / cut_ref[0])
    o_ref[...] = h * env


def _filt2_body(rbf_ref, d2_ref, d20_ref, d21_ref, sc_ref, w1_ref, b1_ref,
                w2_ref, b2_ref, o_ref):
    h = jnp.dot(rbf_ref[...], w1_ref[...], preferred_element_type=jnp.float32)
    h = _ss(h + b1_ref[...])
    h = jnp.dot(h, w2_ref[...], preferred_element_type=jnp.float32)
    h = _ss(h + b2_ref[...])
    c2 = sc_ref[0]
    c1 = sc_ref[1]
    e = sc_ref[2]

    def poly(r):
        rpe = jnp.exp(e * jnp.log(r))
        return 1.0 + rpe * (e * r - (e + 1.0))

    env = poly(d2_ref[...] / c2) * poly(d20_ref[...] / c1) * poly(d21_ref[...] / c1)
    o_ref[...] = h * env


_FBLK = 1600
_FGRID = E1 // _FBLK


def _filter1(rbf1, dist1, cutoff1, fgW1, fgb1, fgW2, fgb2):
    R = rbf1.shape[1]
    return pl.pallas_call(
        _filt1_body,
        grid=(_FGRID,),
        in_specs=[
            pl.BlockSpec((_FBLK, R), lambda i: (i, 0)),
            pl.BlockSpec((_FBLK, 1), lambda i: (i, 0)),
            pl.BlockSpec(memory_space=pltpu.SMEM),
            pl.BlockSpec((R, D), lambda i: (0, 0)),
            pl.BlockSpec((1, D), lambda i: (0, 0)),
            pl.BlockSpec((D, D), lambda i: (0, 0)),
            pl.BlockSpec((1, D), lambda i: (0, 0)),
        ],
        out_specs=pl.BlockSpec((_FBLK, D), lambda i: (i, 0)),
        out_shape=jax.ShapeDtypeStruct((E1, D), jnp.float32),
    )(rbf1, dist1, cutoff1, fgW1, fgb1.reshape(1, D), fgW2, fgb2.reshape(1, D))


def _filter2(rbf2, dist2, dist2_0, dist2_1, scal, fgW1, fgb1, fgW2, fgb2):
    R = rbf2.shape[1]
    return pl.pallas_call(
        _filt2_body,
        grid=(_FGRID,),
        in_specs=[
            pl.BlockSpec((_FBLK, R), lambda i: (i, 0)),
            pl.BlockSpec((_FBLK, 1), lambda i: (i, 0)),
            pl.BlockSpec((_FBLK, 1), lambda i: (i, 0)),
            pl.BlockSpec((_FBLK, 1), lambda i: (i, 0)),
            pl.BlockSpec(memory_space=pltpu.SMEM),
            pl.BlockSpec((R, D), lambda i: (0, 0)),
            pl.BlockSpec((1, D), lambda i: (0, 0)),
            pl.BlockSpec((D, D), lambda i: (0, 0)),
            pl.BlockSpec((1, D), lambda i: (0, 0)),
        ],
        out_specs=pl.BlockSpec((_FBLK, D), lambda i: (i, 0)),
        out_shape=jax.ShapeDtypeStruct((E2, D), jnp.float32),
    )(rbf2, dist2, dist2_0, dist2_1, scal, fgW1, fgb1.reshape(1, D), fgW2,
      fgb2.reshape(1, D))


def _sc_body(dst_hbm, src_hbm, x_hbm, w_hbm, out_hbm,
             dstg, srcg, counts, fill, bk_src, bk_wid, bk_dst,
             xbuf, wbuf, zbuf, accsh, pstarts, semx, semw):
    cid = lax.axis_index("c")
    sid = lax.axis_index("s")
    wrk = cid * 16 + sid
    slab = wrk * EPT
    iota = lax.iota(jnp.int32, 16)
    ones = jnp.ones((16,), jnp.int32)
    zeros16 = jnp.zeros((16,), jnp.int32)

    for q in range(8):
        counts[pl.ds(q * 16, 16)] = zeros16

    def _zb(r, carry):
        for q in range(4):
            zbuf[r, pl.ds(q * 16, 16)] = jnp.zeros((16,), jnp.float32)
        return carry

    lax.fori_loop(0, K, _zb, 0)

    def _p1_outer(i, carry):
        pltpu.sync_copy(dst_hbm.at[pl.ds(slab + i * STG, STG)], dstg)

        def _p1_inner(j, c2):
            d = dstg[pl.ds(j * 16, 16)]
            c = lax.shift_right_logical(d, 14)
            plsc.addupdate_scatter(counts, [c * 16 + iota], ones)
            return c2

        lax.fori_loop(0, STG // 16, _p1_inner, 0)
        return carry

    lax.fori_loop(0, NSTG, _p1_outer, 0)

    def _pf(c, pstart):
        cntv = counts[pl.ds(c * 16, 16)]
        inc = plsc.cumsum(cntv)
        fill[pl.ds(c * 16, 16)] = pstart + (inc - cntv)
        pstarts[c] = pstart
        total = jnp.sum(cntv)
        return (pstart + total + (K - 1)) & (-K)

    pend = lax.fori_loop(0, NCHUNK, _pf, jnp.int32(0))
    pstarts[NCHUNK] = pend

    sent_d = jnp.full((16,), CHUNK, jnp.int32)

    def _sent(r, carry):
        for q in range(K // 16):
            sl = pl.ds(q * 16, 16)
            bk_src[r, sl] = zeros16
            bk_wid[r, sl] = zeros16
            bk_dst[r, 0, sl] = sent_d
        return carry

    lax.fori_loop(0, BROWS, _sent, 0)

    def _p2_outer(i, carry):
        base = slab + i * STG
        pltpu.sync_copy(dst_hbm.at[pl.ds(base, STG)], dstg)
        pltpu.sync_copy(src_hbm.at[pl.ds(base, STG)], srcg)

        def _p2_inner(j, c2):
            d = dstg[pl.ds(j * 16, 16)]
            s = srcg[pl.ds(j * 16, 16)]
            c = lax.shift_right_logical(d, 14)
            dloc = d & jnp.int32(CHUNK - 1)
            eid = base + j * 16 + iota
            wid = jnp.where(eid >= EW, eid - EW, eid)
            ci = c * 16 + iota
            pos = plsc.load_gather(fill, [ci])
            pr = lax.shift_right_logical(pos, 7)
            pc = pos & jnp.int32(K - 1)
            plsc.store_scatter(bk_src, [pr, pc], s)
            plsc.store_scatter(bk_wid, [pr, pc], wid)
            plsc.store_scatter(bk_dst, [pr, zeros16, pc], dloc)
            plsc.addupdate_scatter(fill, [ci], ones)
            return c2

        lax.fori_loop(0, STG // 16, _p2_inner, 0)
        return carry

    lax.fori_loop(0, NSTG, _p2_outer, 0)

    rows0 = sid * ROWS_PER_TILE
    for z in range(ROWS_PER_TILE // K):
        pltpu.sync_copy(zbuf, accsh.at[pl.ds(rows0 + z * K, K)])
    plsc.subcore_barrier()

    def _chunk(c, carry):
        p0 = pstarts[c]
        p1 = pstarts[c + 1]
        nblk = lax.shift_right_logical(p1 - p0, 7)
        r0 = lax.shift_right_logical(p0, 7)

        def _blk(b, c2):
            row = r0 + b
            cpx = pltpu.async_copy(x_hbm.at[bk_src.at[row]], xbuf, semx)
            cpw = pltpu.async_copy(w_hbm.at[bk_wid.at[row]], wbuf, semw)
            cpx.wait()
            cpw.wait()

            def _mul(r, c3):
                for q in range(4):
                    sl = pl.ds(q * 16, 16)
                    xbuf[r, sl] = xbuf[r, sl] * wbuf[r, sl]
                return c3

            lax.fori_loop(0, K, _mul, 0)
            pltpu.sync_copy(xbuf, accsh.at[bk_dst.at[row]], add=True)
            return c2

        lax.fori_loop(0, nblk, _blk, 0)
        plsc.subcore_barrier()

        ob = c * CHUNK + rows0
        pltpu.sync_copy(accsh.at[pl.ds(rows0, ROWS_PER_TILE)],
                        out_hbm.at[cid, pl.ds(ob, ROWS_PER_TILE)])
        for z in range(ROWS_PER_TILE // K):
            pltpu.sync_copy(zbuf, accsh.at[pl.ds(rows0 + z * K, K)])
        plsc.subcore_barrier()
        return carry

    lax.fori_loop(0, NCHUNK, _chunk, 0)


def _sc_scatter(dst, src, x, wcat):
    mesh = plsc.VectorSubcoreMesh(core_axis_name="c", subcore_axis_name="s")
    f = pl.kernel(
        _sc_body,
        mesh=mesh,
        out_type=jax.ShapeDtypeStruct((2, NPAD, D), jnp.float32),
        scratch_types=[
            pltpu.VMEM((STG,), jnp.int32),
            pltpu.VMEM((STG,), jnp.int32),
            pltpu.VMEM((128,), jnp.int32),
            pltpu.VMEM((128,), jnp.int32),
            pltpu.VMEM((BROWS, K), jnp.int32),
            pltpu.VMEM((BROWS, K), jnp.int32),
            pltpu.VMEM((BROWS, 1, K), jnp.int32),
            pltpu.VMEM((K, D), jnp.float32),
            pltpu.VMEM((K, D), jnp.float32),
            pltpu.VMEM((K, D), jnp.float32),
            pltpu.VMEM_SHARED((CHUNK + K, D), jnp.float32),
            pltpu.SMEM((NCHUNK + 1,), jnp.int32),
            pltpu.SemaphoreType.DMA,
            pltpu.SemaphoreType.DMA,
        ],
    )
    return f(dst, src, x, wcat)


def _final_body(pa_ref, pb_ref, x0_ref, w2_ref, b2_ref, w3_ref, b3_ref, o_ref):
    s = pa_ref[0] + pb_ref[0]
    h = jnp.dot(s, w2_ref[...], preferred_element_type=jnp.float32)
    h = _ss(h + b2_ref[...])
    y = jnp.dot(h, w3_ref[...], preferred_element_type=jnp.float32)
    o_ref[...] = y + b3_ref[...] + x0_ref[...]


def _final(partial, x0, W2, b2, W3, b3):
    blk = 1000
    grid = NTOT // blk
    return pl.pallas_call(
        _final_body,
        grid=(grid,),
        in_specs=[
            pl.BlockSpec((1, blk, D), lambda i: (0, i, 0)),
            pl.BlockSpec((1, blk, D), lambda i: (1, i, 0)),
            pl.BlockSpec((blk, D), lambda i: (i, 0)),
            pl.BlockSpec((D, D), lambda i: (0, 0)),
            pl.BlockSpec((1, D), lambda i: (0, 0)),
            pl.BlockSpec((D, D), lambda i: (0, 0)),
            pl.BlockSpec((1, D), lambda i: (0, 0)),
        ],
        out_specs=pl.BlockSpec((blk, D), lambda i: (i, 0)),
        out_shape=jax.ShapeDtypeStruct((NTOT, D), jnp.float32),
    )(partial, partial, x0, W2, b2.reshape(1, D), W3, b3.reshape(1, D))


def kernel(edge_index1, edge_index2, node_feature, node_feature_ghost,
           rbf_tensor1, dist1, rbf_tensor2, dist2, cutoff1, cutoff2, exponent,
           dist2_0, dist2_1, W1, b1, W2, b2, W3, b3, fgW1, fgb1, fgW2, fgb2):
    x0 = jnp.concatenate([node_feature, node_feature_ghost], axis=0)
    x = _lin1(x0, W1, b1)

    w1e = _filter1(rbf_tensor1, dist1, cutoff1, fgW1, fgb1, fgW2, fgb2)
    scal = jnp.concatenate(
        [cutoff2, cutoff1, jnp.asarray(exponent, jnp.float32).reshape(1)])
    w2e = _filter2(rbf_tensor2, dist2, dist2_0, dist2_1, scal, fgW1, fgb1,
                   fgW2, fgb2)
    wcat = jnp.concatenate([w1e, w2e], axis=0)

    dst = jnp.concatenate([edge_index1[0], edge_index2[2],
                           edge_index1[2], edge_index2[3]])
    src = jnp.concatenate([edge_index1[2], edge_index2[3],
                           edge_index1[0], edge_index2[2]])

    partial = _sc_scatter(dst, src, x, wcat)

    y = _final(partial, x0, W2, b2, W3, b3)
    return (y[:N1], y[N1:])


# speedup vs baseline: 1.6921x; 1.6921x over previous
import jax
import jax.numpy as jnp
from jax import lax
from jax.experimental import pallas as pl
from jax.experimental.pallas import tpu as pltpu
from jax.experimental.pallas import tpu_sc as plsc

N1 = 50000
NG = 50000
NTOT = N1 + NG
E1 = 160000
E2 = 160000
EW = E1 + E2
ETOT = 2 * EW
D = 64
PI = 3.14159265
LN2 = 0.6931471805599453

NTILES = 32
EPT = ETOT // NTILES
STG = 2000
NSTG = EPT // STG
CHUNK = 8192
NCHUNK = 13
CSHIFT = 13
NPAD = NCHUNK * CHUNK
K = 128
BROWS = (EPT + NCHUNK * (K - 1) + K - 1) // K
ROWS_PER_TILE = CHUNK // 16


def _ss(h):
    return jnp.log(jnp.exp(h) + 1.0) - LN2


def _lin1_body(xin_ref, w_ref, b_ref, o_ref):
    o_ref[...] = (
        jnp.dot(xin_ref[...], w_ref[...], preferred_element_type=jnp.float32)
        + b_ref[...]
    )


def _lin1(xin, W1, b1):
    blk = 1000
    grid = NTOT // blk
    return pl.pallas_call(
        _lin1_body,
        grid=(grid,),
        in_specs=[
            pl.BlockSpec((blk, D), lambda i: (i, 0)),
            pl.BlockSpec((D, D), lambda i: (0, 0)),
            pl.BlockSpec((1, D), lambda i: (0, 0)),
        ],
        out_specs=pl.BlockSpec((blk, D), lambda i: (i, 0)),
        out_shape=jax.ShapeDtypeStruct((NTOT, D), jnp.float32),
    )(xin, W1, b1.reshape(1, D))


def _filt1_body(rbf_ref, dist_ref, cut_ref, w1_ref, b1_ref, w2_ref, b2_ref, o_ref):
    h = jnp.dot(rbf_ref[...], w1_ref[...], preferred_element_type=jnp.float32)
    h = _ss(h + b1_ref[...])
    h = jnp.dot(h, w2_ref[...], preferred_element_type=jnp.float32)
    h = _ss(h + b2_ref[...])
    env = 1.0 + jnp.cos(PI * dist_ref[...] / cut_ref[0])
    o_ref[...] = h * env


def _filt2_body(rbf_ref, d2_ref, d20_ref, d21_ref, sc_ref, w1_ref, b1_ref,
                w2_ref, b2_ref, o_ref):
    h = jnp.dot(rbf_ref[...], w1_ref[...], preferred_element_type=jnp.float32)
    h = _ss(h + b1_ref[...])
    h = jnp.dot(h, w2_ref[...], preferred_element_type=jnp.float32)
    h = _ss(h + b2_ref[...])
    c2 = sc_ref[0]
    c1 = sc_ref[1]
    e = sc_ref[2]

    def poly(r):
        rpe = jnp.exp(e * jnp.log(r))
        return 1.0 + rpe * (e * r - (e + 1.0))

    env = poly(d2_ref[...] / c2) * poly(d20_ref[...] / c1) * poly(d21_ref[...] / c1)
    o_ref[...] = h * env


_FBLK = 1600
_FGRID = E1 // _FBLK


def _filter1(rbf1, dist1, cutoff1, fgW1, fgb1, fgW2, fgb2):
    R = rbf1.shape[1]
    return pl.pallas_call(
        _filt1_body,
        grid=(_FGRID,),
        in_specs=[
            pl.BlockSpec((_FBLK, R), lambda i: (i, 0)),
            pl.BlockSpec((_FBLK, 1), lambda i: (i, 0)),
            pl.BlockSpec(memory_space=pltpu.SMEM),
            pl.BlockSpec((R, D), lambda i: (0, 0)),
            pl.BlockSpec((1, D), lambda i: (0, 0)),
            pl.BlockSpec((D, D), lambda i: (0, 0)),
            pl.BlockSpec((1, D), lambda i: (0, 0)),
        ],
        out_specs=pl.BlockSpec((_FBLK, D), lambda i: (i, 0)),
        out_shape=jax.ShapeDtypeStruct((E1, D), jnp.float32),
    )(rbf1, dist1, cutoff1, fgW1, fgb1.reshape(1, D), fgW2, fgb2.reshape(1, D))


def _filter2(rbf2, dist2, dist2_0, dist2_1, scal, fgW1, fgb1, fgW2, fgb2):
    R = rbf2.shape[1]
    return pl.pallas_call(
        _filt2_body,
        grid=(_FGRID,),
        in_specs=[
            pl.BlockSpec((_FBLK, R), lambda i: (i, 0)),
            pl.BlockSpec((_FBLK, 1), lambda i: (i, 0)),
            pl.BlockSpec((_FBLK, 1), lambda i: (i, 0)),
            pl.BlockSpec((_FBLK, 1), lambda i: (i, 0)),
            pl.BlockSpec(memory_space=pltpu.SMEM),
            pl.BlockSpec((R, D), lambda i: (0, 0)),
            pl.BlockSpec((1, D), lambda i: (0, 0)),
            pl.BlockSpec((D, D), lambda i: (0, 0)),
            pl.BlockSpec((1, D), lambda i: (0, 0)),
        ],
        out_specs=pl.BlockSpec((_FBLK, D), lambda i: (i, 0)),
        out_shape=jax.ShapeDtypeStruct((E2, D), jnp.float32),
    )(rbf2, dist2, dist2_0, dist2_1, scal, fgW1, fgb1.reshape(1, D), fgW2,
      fgb2.reshape(1, D))


def _sc_body(dst_hbm, src_hbm, x_hbm, w_hbm, out_hbm,
             dstg, srcg, counts, fill, bk_src, bk_wid, bk_dst,
             xbuf, wbuf, zbuf, accsh, pstarts, semx, semw):
    cid = lax.axis_index("c")
    sid = lax.axis_index("s")
    wrk = cid * 16 + sid
    slab = wrk * EPT
    iota = lax.iota(jnp.int32, 16)
    ones = jnp.ones((16,), jnp.int32)
    zeros16 = jnp.zeros((16,), jnp.int32)

    for q in range(16):
        counts[pl.ds(q * 16, 16)] = zeros16

    def _zb(r, carry):
        for q in range(4):
            zbuf[r, pl.ds(q * 16, 16)] = jnp.zeros((16,), jnp.float32)
        return carry

    lax.fori_loop(0, K, _zb, 0)

    def _p1_outer(i, carry):
        pltpu.sync_copy(dst_hbm.at[pl.ds(slab + i * STG, STG)], dstg)

        def _p1_inner(j, c2):
            d = dstg[pl.ds(j * 16, 16)]
            c = lax.shift_right_logical(d, CSHIFT)
            plsc.addupdate_scatter(counts, [c * 16 + iota], ones)
            return c2

        lax.fori_loop(0, STG // 16, _p1_inner, 0)
        return carry

    lax.fori_loop(0, NSTG, _p1_outer, 0)

    def _pf(c, pstart):
        cntv = counts[pl.ds(c * 16, 16)]
        inc = plsc.cumsum(cntv)
        fill[pl.ds(c * 16, 16)] = pstart + (inc - cntv)
        pstarts[c] = pstart
        total = jnp.sum(cntv)
        return (pstart + total + (K - 1)) & (-K)

    pend = lax.fori_loop(0, NCHUNK, _pf, jnp.int32(0))
    pstarts[NCHUNK] = pend

    sent_d = jnp.full((16,), CHUNK, jnp.int32)

    def _sent(r, carry):
        for q in range(K // 16):
            sl = pl.ds(q * 16, 16)
            bk_src[r, sl] = zeros16
            bk_wid[r, sl] = zeros16
            bk_dst[r, sl] = sent_d
        return carry

    lax.fori_loop(0, BROWS, _sent, 0)

    def _p2_outer(i, carry):
        base = slab + i * STG
        pltpu.sync_copy(dst_hbm.at[pl.ds(base, STG)], dstg)
        pltpu.sync_copy(src_hbm.at[pl.ds(base, STG)], srcg)

        def _p2_inner(j, c2):
            d = dstg[pl.ds(j * 16, 16)]
            s = srcg[pl.ds(j * 16, 16)]
            c = lax.shift_right_logical(d, CSHIFT)
            dloc = d & jnp.int32(CHUNK - 1)
            eid = base + j * 16 + iota
            wid = jnp.where(eid >= EW, eid - EW, eid)
            ci = c * 16 + iota
            pos = plsc.load_gather(fill, [ci])
            pr = lax.shift_right_logical(pos, 7)
            pc = pos & jnp.int32(K - 1)
            plsc.store_scatter(bk_src, [pr, pc], s)
            plsc.store_scatter(bk_wid, [pr, pc], wid)
            plsc.store_scatter(bk_dst, [pr, pc], dloc)
            plsc.addupdate_scatter(fill, [ci], ones)
            return c2

        lax.fori_loop(0, STG // 16, _p2_inner, 0)
        return carry

    lax.fori_loop(0, NSTG, _p2_outer, 0)

    rows0 = sid * ROWS_PER_TILE
    for z in range(ROWS_PER_TILE // K):
        pltpu.sync_copy(zbuf, accsh.at[pl.ds(rows0 + z * K, K)])
    plsc.subcore_barrier()

    def _chunk(c, carry):
        p0 = pstarts[c]
        p1 = pstarts[c + 1]
        nblk = lax.shift_right_logical(p1 - p0, 7)
        r0 = lax.shift_right_logical(p0, 7)

        def _blk(b, c2):
            row = r0 + b
            cpx = pltpu.async_copy(x_hbm.at[bk_src.at[row]], xbuf, semx)
            cpw = pltpu.async_copy(w_hbm.at[bk_wid.at[row]], wbuf, semw)
            cpx.wait()
            cpw.wait()

            def _mul(r, c3):
                for q in range(4):
                    sl = pl.ds(q * 16, 16)
                    xbuf[r, sl] = xbuf[r, sl] * wbuf[r, sl]
                return c3

            lax.fori_loop(0, K, _mul, 0)
            pltpu.sync_copy(xbuf, accsh.at[bk_dst.at[row]], add=True)
            return c2

        lax.fori_loop(0, nblk, _blk, 0)
        plsc.subcore_barrier()

        ob = c * CHUNK + rows0
        pltpu.sync_copy(accsh.at[pl.ds(rows0, ROWS_PER_TILE)],
                        out_hbm.at[cid, pl.ds(ob, ROWS_PER_TILE)])
        for z in range(ROWS_PER_TILE // K):
            pltpu.sync_copy(zbuf, accsh.at[pl.ds(rows0 + z * K, K)])
        plsc.subcore_barrier()
        return carry

    lax.fori_loop(0, NCHUNK, _chunk, 0)


def _sc_scatter(dst, src, x, wcat):
    mesh = plsc.VectorSubcoreMesh(core_axis_name="c", subcore_axis_name="s")
    f = pl.kernel(
        _sc_body,
        mesh=mesh,
        compiler_params=pltpu.CompilerParams(
            needs_layout_passes=False, use_tc_tiling_on_sc=False),
        out_type=jax.ShapeDtypeStruct((2, NPAD, D), jnp.float32),
        scratch_types=[
            pltpu.VMEM((STG,), jnp.int32),
            pltpu.VMEM((STG,), jnp.int32),
            pltpu.VMEM((256,), jnp.int32),
            pltpu.VMEM((256,), jnp.int32),
            pltpu.VMEM((BROWS, K), jnp.int32),
            pltpu.VMEM((BROWS, K), jnp.int32),
            pltpu.VMEM((BROWS, K), jnp.int32),
            pltpu.VMEM((K, D), jnp.float32),
            pltpu.VMEM((K, D), jnp.float32),
            pltpu.VMEM((K, D), jnp.float32),
            pltpu.VMEM_SHARED((CHUNK + K, D), jnp.float32),
            pltpu.SMEM((NCHUNK + 1,), jnp.int32),
            pltpu.SemaphoreType.DMA,
            pltpu.SemaphoreType.DMA,
        ],
    )
    return f(dst, src, x, wcat)


def _final_body(pa_ref, pb_ref, x0_ref, w2_ref, b2_ref, w3_ref, b3_ref, o_ref):
    s = pa_ref[0] + pb_ref[0]
    h = jnp.dot(s, w2_ref[...], preferred_element_type=jnp.float32)
    h = _ss(h + b2_ref[...])
    y = jnp.dot(h, w3_ref[...], preferred_element_type=jnp.float32)
    o_ref[...] = y + b3_ref[...] + x0_ref[...]


def _final(partial, x0, W2, b2, W3, b3):
    blk = 1000
    grid = NTOT // blk
    return pl.pallas_call(
        _final_body,
        grid=(grid,),
        in_specs=[
            pl.BlockSpec((1, blk, D), lambda i: (0, i, 0)),
            pl.BlockSpec((1, blk, D), lambda i: (1, i, 0)),
            pl.BlockSpec((blk, D), lambda i: (i, 0)),
            pl.BlockSpec((D, D), lambda i: (0, 0)),
            pl.BlockSpec((1, D), lambda i: (0, 0)),
            pl.BlockSpec((D, D), lambda i: (0, 0)),
            pl.BlockSpec((1, D), lambda i: (0, 0)),
        ],
        out_specs=pl.BlockSpec((blk, D), lambda i: (i, 0)),
        out_shape=jax.ShapeDtypeStruct((NTOT, D), jnp.float32),
    )(partial, partial, x0, W2, b2.reshape(1, D), W3, b3.reshape(1, D))


def kernel(edge_index1, edge_index2, node_feature, node_feature_ghost,
           rbf_tensor1, dist1, rbf_tensor2, dist2, cutoff1, cutoff2, exponent,
           dist2_0, dist2_1, W1, b1, W2, b2, W3, b3, fgW1, fgb1, fgW2, fgb2):
    x0 = jnp.concatenate([node_feature, node_feature_ghost], axis=0)
    x = _lin1(x0, W1, b1)

    w1e = _filter1(rbf_tensor1, dist1, cutoff1, fgW1, fgb1, fgW2, fgb2)
    scal = jnp.concatenate(
        [cutoff2, cutoff1, jnp.asarray(exponent, jnp.float32).reshape(1)])
    w2e = _filter2(rbf_tensor2, dist2, dist2_0, dist2_1, scal, fgW1, fgb1,
                   fgW2, fgb2)
    wcat = jnp.concatenate([w1e, w2e], axis=0)

    dst = jnp.concatenate([edge_index1[0], edge_index2[2],
                           edge_index1[2], edge_index2[3]])
    src = jnp.concatenate([edge_index1[2], edge_index2[3],
                           edge_index1[0], edge_index2[2]])

    partial = _sc_scatter(dst, src, x, wcat)

    y = _final(partial, x0, W2, b2, W3, b3)
    return (y[:N1], y[N1:])


# speedup vs baseline: 2.3621x; 1.3960x over previous
import jax
import jax.numpy as jnp
from jax import lax
from jax.experimental import pallas as pl
from jax.experimental.pallas import tpu as pltpu
from jax.experimental.pallas import tpu_sc as plsc

N1 = 50000
NG = 50000
NTOT = N1 + NG
E1 = 160000
E2 = 160000
EW = E1 + E2
ETOT = 2 * EW
D = 64
PI = 3.14159265
LN2 = 0.6931471805599453

NTILES = 32
EPT = ETOT // NTILES
STG = 2000
NSTG = EPT // STG
CHUNK = 8192
NCHUNK = 13
CSHIFT = 13
NPAD = NCHUNK * CHUNK
K = 128
BROWS = (EPT + NCHUNK * (K - 1) + K - 1) // K
ROWS_PER_TILE = CHUNK // 16


def _ss(h):
    return jnp.log(jnp.exp(h) + 1.0) - LN2


def _lin1_body(a_ref, b_ref, w_ref, bias_ref, o_ref):
    i = pl.program_id(0)
    nb = pl.num_programs(0) // 2

    @pl.when(i < nb)
    def _():
        o_ref[...] = (
            jnp.dot(a_ref[...], w_ref[...], preferred_element_type=jnp.float32)
            + bias_ref[...]
        )

    @pl.when(i >= nb)
    def _():
        o_ref[...] = (
            jnp.dot(b_ref[...], w_ref[...], preferred_element_type=jnp.float32)
            + bias_ref[...]
        )


def _lin1(nf, nfg, W1, b1):
    blk = 1000
    nb = N1 // blk
    return pl.pallas_call(
        _lin1_body,
        grid=(2 * nb,),
        in_specs=[
            pl.BlockSpec((blk, D), lambda i: (jnp.minimum(i, nb - 1), 0)),
            pl.BlockSpec((blk, D), lambda i: (jnp.maximum(i - nb, 0), 0)),
            pl.BlockSpec((D, D), lambda i: (0, 0)),
            pl.BlockSpec((1, D), lambda i: (0, 0)),
        ],
        out_specs=pl.BlockSpec((blk, D), lambda i: (i, 0)),
        out_shape=jax.ShapeDtypeStruct((NTOT, D), jnp.float32),
    )(nf, nfg, W1, b1.reshape(1, D))


_FBLK = 1280
_FGRID = E1 // _FBLK


def _filt_body(r1_ref, r2_ref, e1_ref, e2_ref, w1t_ref, b1_ref, w2_ref,
               b2_ref, o_ref):
    i = pl.program_id(0)

    def compute(rref, eref):
        h = lax.dot_general(rref[...], w1t_ref[...], (((0,), (1,)), ((), ())),
                            preferred_element_type=jnp.float32)
        h = _ss(h + b1_ref[...])
        h = jnp.dot(h, w2_ref[...], preferred_element_type=jnp.float32)
        h = _ss(h + b2_ref[...])
        o_ref[...] = h * eref[...]

    @pl.when(i < _FGRID)
    def _():
        compute(r1_ref, e1_ref)

    @pl.when(i >= _FGRID)
    def _():
        compute(r2_ref, e2_ref)


def _filters(rbf1_t, rbf2_t, env1, env2, fgW1_t, fgb1, fgW2, fgb2):
    R = rbf1_t.shape[0]
    return pl.pallas_call(
        _filt_body,
        grid=(2 * _FGRID,),
        in_specs=[
            pl.BlockSpec((R, _FBLK), lambda i: (0, jnp.minimum(i, _FGRID - 1))),
            pl.BlockSpec((R, _FBLK), lambda i: (0, jnp.maximum(i - _FGRID, 0))),
            pl.BlockSpec((_FBLK, 1), lambda i: (jnp.minimum(i, _FGRID - 1), 0)),
            pl.BlockSpec((_FBLK, 1), lambda i: (jnp.maximum(i - _FGRID, 0), 0)),
            pl.BlockSpec((D, R), lambda i: (0, 0)),
            pl.BlockSpec((1, D), lambda i: (0, 0)),
            pl.BlockSpec((D, D), lambda i: (0, 0)),
            pl.BlockSpec((1, D), lambda i: (0, 0)),
        ],
        out_specs=pl.BlockSpec((_FBLK, D), lambda i: (i, 0)),
        out_shape=jax.ShapeDtypeStruct((EW, D), jnp.float32),
    )(rbf1_t, rbf2_t, env1, env2, fgW1_t, fgb1.reshape(1, D), fgW2,
      fgb2.reshape(1, D))


def _sc_body(dst_hbm, src_hbm, x_hbm, w_hbm, out_hbm,
             dstg, srcg, counts, fill, bk_src, bk_wid, bk_dst,
             xbuf, wbuf, zbuf, accsh, pstarts, semx, semw):
    cid = lax.axis_index("c")
    sid = lax.axis_index("s")
    wrk = cid * 16 + sid
    slab = wrk * EPT
    iota = lax.iota(jnp.int32, 16)
    ones = jnp.ones((16,), jnp.int32)
    zeros16 = jnp.zeros((16,), jnp.int32)

    for q in range(16):
        counts[pl.ds(q * 16, 16)] = zeros16

    def _zb(r, carry):
        for q in range(4):
            zbuf[r, pl.ds(q * 16, 16)] = jnp.zeros((16,), jnp.float32)
        return carry

    lax.fori_loop(0, K, _zb, 0)

    def _p1_outer(i, carry):
        pltpu.sync_copy(dst_hbm.at[pl.ds(slab + i * STG, STG)], dstg)

        def _p1_inner(j, c2):
            d = dstg[pl.ds(j * 16, 16)]
            c = lax.shift_right_logical(d, CSHIFT)
            plsc.addupdate_scatter(counts, [c * 16 + iota], ones)
            return c2

        lax.fori_loop(0, STG // 16, _p1_inner, 0)
        return carry

    lax.fori_loop(0, NSTG, _p1_outer, 0)

    def _pf(c, pstart):
        cntv = counts[pl.ds(c * 16, 16)]
        inc = plsc.cumsum(cntv)
        fill[pl.ds(c * 16, 16)] = pstart + (inc - cntv)
        pstarts[c] = pstart
        total = jnp.sum(cntv)
        return (pstart + total + (K - 1)) & (-K)

    pend = lax.fori_loop(0, NCHUNK, _pf, jnp.int32(0))
    pstarts[NCHUNK] = pend

    sent_d = jnp.full((16,), CHUNK, jnp.int32)

    def _sent(r, carry):
        for q in range(K // 16):
            sl = pl.ds(q * 16, 16)
            bk_src[r, sl] = zeros16
            bk_wid[r, sl] = zeros16
            bk_dst[r, sl] = sent_d
        return carry

    lax.fori_loop(0, BROWS, _sent, 0)

    def _p2_outer(i, carry):
        base = slab + i * STG
        pltpu.sync_copy(dst_hbm.at[pl.ds(base, STG)], dstg)
        pltpu.sync_copy(src_hbm.at[pl.ds(base, STG)], srcg)

        def _p2_inner(j, c2):
            d = dstg[pl.ds(j * 16, 16)]
            s = srcg[pl.ds(j * 16, 16)]
            c = lax.shift_right_logical(d, CSHIFT)
            dloc = d & jnp.int32(CHUNK - 1)
            eid = base + j * 16 + iota
            wid = jnp.where(eid >= EW, eid - EW, eid)
            ci = c * 16 + iota
            pos = plsc.load_gather(fill, [ci])
            pr = lax.shift_right_logical(pos, 7)
            pc = pos & jnp.int32(K - 1)
            plsc.store_scatter(bk_src, [pr, pc], s)
            plsc.store_scatter(bk_wid, [pr, pc], wid)
            plsc.store_scatter(bk_dst, [pr, pc], dloc)
            plsc.addupdate_scatter(fill, [ci], ones)
            return c2

        lax.fori_loop(0, STG // 16, _p2_inner, 0)
        return carry

    lax.fori_loop(0, NSTG, _p2_outer, 0)

    rows0 = sid * ROWS_PER_TILE
    for z in range(ROWS_PER_TILE // K):
        pltpu.sync_copy(zbuf, accsh.at[pl.ds(rows0 + z * K, K)])
    plsc.subcore_barrier()

    def _chunk(c, carry):
        p0 = pstarts[c]
        p1 = pstarts[c + 1]
        nblk = lax.shift_right_logical(p1 - p0, 7)
        r0 = lax.shift_right_logical(p0, 7)

        def _blk(b, c2):
            row = r0 + b
            cpx = pltpu.async_copy(x_hbm.at[bk_src.at[row]], xbuf, semx)
            cpw = pltpu.async_copy(w_hbm.at[bk_wid.at[row]], wbuf, semw)
            cpx.wait()
            cpw.wait()

            def _mul(r, c3):
                for q in range(4):
                    sl = pl.ds(q * 16, 16)
                    xbuf[r, sl] = xbuf[r, sl] * wbuf[r, sl]
                return c3

            lax.fori_loop(0, K, _mul, 0)
            pltpu.sync_copy(xbuf, accsh.at[bk_dst.at[row]], add=True)
            return c2

        lax.fori_loop(0, nblk, _blk, 0)
        plsc.subcore_barrier()

        ob = c * CHUNK + rows0
        pltpu.sync_copy(accsh.at[pl.ds(rows0, ROWS_PER_TILE)],
                        out_hbm.at[cid, pl.ds(ob, ROWS_PER_TILE)])
        for z in range(ROWS_PER_TILE // K):
            pltpu.sync_copy(zbuf, accsh.at[pl.ds(rows0 + z * K, K)])
        plsc.subcore_barrier()
        return carry

    lax.fori_loop(0, NCHUNK, _chunk, 0)


def _sc_scatter(dst, src, x, wcat):
    mesh = plsc.VectorSubcoreMesh(core_axis_name="c", subcore_axis_name="s")
    f = pl.kernel(
        _sc_body,
        mesh=mesh,
        compiler_params=pltpu.CompilerParams(
            needs_layout_passes=False, use_tc_tiling_on_sc=False),
        out_type=jax.ShapeDtypeStruct((2, NPAD, D), jnp.float32),
        scratch_types=[
            pltpu.VMEM((STG,), jnp.int32),
            pltpu.VMEM((STG,), jnp.int32),
            pltpu.VMEM((256,), jnp.int32),
            pltpu.VMEM((256,), jnp.int32),
            pltpu.VMEM((BROWS, K), jnp.int32),
            pltpu.VMEM((BROWS, K), jnp.int32),
            pltpu.VMEM((BROWS, K), jnp.int32),
            pltpu.VMEM((K, D), jnp.float32),
            pltpu.VMEM((K, D), jnp.float32),
            pltpu.VMEM((K, D), jnp.float32),
            pltpu.VMEM_SHARED((CHUNK + K, D), jnp.float32),
            pltpu.SMEM((NCHUNK + 1,), jnp.int32),
            pltpu.SemaphoreType.DMA,
            pltpu.SemaphoreType.DMA,
        ],
    )
    return f(dst, src, x, wcat)


def _final_body(pa_ref, pb_ref, a_ref, b_ref, w2_ref, b2_ref, w3_ref, b3_ref,
                o1_ref, o2_ref):
    i = pl.program_id(0)
    nb = pl.num_programs(0) // 2
    s = pa_ref[0] + pb_ref[0]
    h = jnp.dot(s, w2_ref[...], preferred_element_type=jnp.float32)
    h = _ss(h + b2_ref[...])
    y = jnp.dot(h, w3_ref[...], preferred_element_type=jnp.float32)
    y = y + b3_ref[...]

    @pl.when(i < nb)
    def _():
        o1_ref[...] = y + a_ref[...]

    @pl.when(i >= nb)
    def _():
        o2_ref[...] = y + b_ref[...]


def _final(partial, nf, nfg, W2, b2, W3, b3):
    blk = 1000
    nb = N1 // blk
    return pl.pallas_call(
        _final_body,
        grid=(2 * nb,),
        in_specs=[
            pl.BlockSpec((1, blk, D), lambda i: (0, i, 0)),
            pl.BlockSpec((1, blk, D), lambda i: (1, i, 0)),
            pl.BlockSpec((blk, D), lambda i: (jnp.minimum(i, nb - 1), 0)),
            pl.BlockSpec((blk, D), lambda i: (jnp.maximum(i - nb, 0), 0)),
            pl.BlockSpec((D, D), lambda i: (0, 0)),
            pl.BlockSpec((1, D), lambda i: (0, 0)),
            pl.BlockSpec((D, D), lambda i: (0, 0)),
            pl.BlockSpec((1, D), lambda i: (0, 0)),
        ],
        out_specs=[
            pl.BlockSpec((blk, D), lambda i: (jnp.minimum(i, nb - 1), 0)),
            pl.BlockSpec((blk, D), lambda i: (jnp.maximum(i - nb, 0), 0)),
        ],
        out_shape=[
            jax.ShapeDtypeStruct((N1, D), jnp.float32),
            jax.ShapeDtypeStruct((NG, D), jnp.float32),
        ],
    )(partial, partial, nf, nfg, W2, b2.reshape(1, D), W3, b3.reshape(1, D))


def kernel(edge_index1, edge_index2, node_feature, node_feature_ghost,
           rbf_tensor1, dist1, rbf_tensor2, dist2, cutoff1, cutoff2, exponent,
           dist2_0, dist2_1, W1, b1, W2, b2, W3, b3, fgW1, fgb1, fgW2, fgb2):
    x = _lin1(node_feature, node_feature_ghost, W1, b1)

    dshape = (E1 // 128, 128)
    env1d = 1.0 + jnp.cos(PI * dist1.reshape(dshape) / cutoff1)
    env1 = lax.optimization_barrier(env1d).reshape(E1, 1)
    ef = jnp.asarray(exponent, jnp.float32)

    def _poly(r):
        rpe = r ** ef
        return 1.0 + rpe * (ef * r - (ef + 1.0))

    env2d = (_poly(dist2.reshape(dshape) / cutoff2)
             * _poly(dist2_0.reshape(dshape) / cutoff1)
             * _poly(dist2_1.reshape(dshape) / cutoff1))
    env2 = lax.optimization_barrier(env2d).reshape(E2, 1)

    wcat = _filters(rbf_tensor1.T, rbf_tensor2.T, env1, env2, fgW1.T, fgb1,
                    fgW2, fgb2)

    dst = jnp.concatenate([edge_index1[0], edge_index2[2],
                           edge_index1[2], edge_index2[3]])
    src = jnp.concatenate([edge_index1[2], edge_index2[3],
                           edge_index1[0], edge_index2[2]])

    partial = _sc_scatter(dst, src, x, wcat)

    return _final(partial, node_feature, node_feature_ghost, W2, b2, W3, b3)


# speedup vs baseline: 2.5350x; 1.0732x over previous
import jax
import jax.numpy as jnp
from jax import lax
from jax.experimental import pallas as pl
from jax.experimental.pallas import tpu as pltpu
from jax.experimental.pallas import tpu_sc as plsc

N1 = 50000
NG = 50000
NTOT = N1 + NG
E1 = 160000
E2 = 160000
EW = E1 + E2
ETOT = 2 * EW
D = 64
PI = 3.14159265
LN2 = 0.6931471805599453

NTILES = 32
EPT = ETOT // NTILES
STG = 2000
NSTG = EPT // STG
CHUNK = 8192
NCHUNK = 13
CSHIFT = 13
NPAD = NCHUNK * CHUNK
K = 128
BROWS = (EPT + NCHUNK * (K - 1) + K - 1) // K
ROWS_PER_TILE = CHUNK // 16


def _ss(h):
    return jnp.log(jnp.exp(h) + 1.0) - LN2


def _lin1_body(a_ref, b_ref, w_ref, bias_ref, o_ref):
    i = pl.program_id(0)
    nb = pl.num_programs(0) // 2

    @pl.when(i < nb)
    def _():
        o_ref[...] = (
            jnp.dot(a_ref[...], w_ref[...], preferred_element_type=jnp.float32)
            + bias_ref[...]
        )

    @pl.when(i >= nb)
    def _():
        o_ref[...] = (
            jnp.dot(b_ref[...], w_ref[...], preferred_element_type=jnp.float32)
            + bias_ref[...]
        )


def _lin1(nf, nfg, W1, b1):
    blk = 1000
    nb = N1 // blk
    return pl.pallas_call(
        _lin1_body,
        grid=(2 * nb,),
        in_specs=[
            pl.BlockSpec((blk, D), lambda i: (jnp.minimum(i, nb - 1), 0)),
            pl.BlockSpec((blk, D), lambda i: (jnp.maximum(i - nb, 0), 0)),
            pl.BlockSpec((D, D), lambda i: (0, 0)),
            pl.BlockSpec((1, D), lambda i: (0, 0)),
        ],
        out_specs=pl.BlockSpec((blk, D), lambda i: (i, 0)),
        out_shape=jax.ShapeDtypeStruct((NTOT, D), jnp.float32),
    )(nf, nfg, W1, b1.reshape(1, D))


_FBLK = 1280
_FGRID = E1 // _FBLK


def _filt_body(r1_ref, r2_ref, e1_ref, e2_ref, w1t_ref, b1_ref, w2_ref,
               b2_ref, o_ref):
    i = pl.program_id(0)

    def compute(rref, eref):
        h = lax.dot_general(rref[...], w1t_ref[...], (((0,), (1,)), ((), ())),
                            preferred_element_type=jnp.float32)
        h = _ss(h + b1_ref[...])
        h = jnp.dot(h, w2_ref[...], preferred_element_type=jnp.float32)
        h = _ss(h + b2_ref[...])
        o_ref[...] = h * eref[...]

    @pl.when(i < _FGRID)
    def _():
        compute(r1_ref, e1_ref)

    @pl.when(i >= _FGRID)
    def _():
        compute(r2_ref, e2_ref)


def _filters(rbf1_t, rbf2_t, env1, env2, fgW1_t, fgb1, fgW2, fgb2):
    R = rbf1_t.shape[0]
    return pl.pallas_call(
        _filt_body,
        grid=(2 * _FGRID,),
        in_specs=[
            pl.BlockSpec((R, _FBLK), lambda i: (0, jnp.minimum(i, _FGRID - 1))),
            pl.BlockSpec((R, _FBLK), lambda i: (0, jnp.maximum(i - _FGRID, 0))),
            pl.BlockSpec((_FBLK, 1), lambda i: (jnp.minimum(i, _FGRID - 1), 0)),
            pl.BlockSpec((_FBLK, 1), lambda i: (jnp.maximum(i - _FGRID, 0), 0)),
            pl.BlockSpec((D, R), lambda i: (0, 0)),
            pl.BlockSpec((1, D), lambda i: (0, 0)),
            pl.BlockSpec((D, D), lambda i: (0, 0)),
            pl.BlockSpec((1, D), lambda i: (0, 0)),
        ],
        out_specs=pl.BlockSpec((_FBLK, D), lambda i: (i, 0)),
        out_shape=jax.ShapeDtypeStruct((EW, D), jnp.float32),
    )(rbf1_t, rbf2_t, env1, env2, fgW1_t, fgb1.reshape(1, D), fgW2,
      fgb2.reshape(1, D))


def _sc_body(dst_hbm, src_hbm, x_hbm, w_hbm, out_hbm,
             dstg, srcg, counts, fill, bk_sd, bk_wid,
             idx0, idx1, dtmp, xbuf, wbuf, xbuf1, wbuf1, zbuf, accsh, pstarts,
             semx, semw, semx1, semw1):
    cid = lax.axis_index("c")
    sid = lax.axis_index("s")
    wrk = cid * 16 + sid
    slab = wrk * EPT
    iota = lax.iota(jnp.int32, 16)
    ones = jnp.ones((16,), jnp.int32)
    zeros16 = jnp.zeros((16,), jnp.int32)

    for q in range(16):
        counts[pl.ds(q * 16, 16)] = zeros16

    def _zb(r, carry):
        for q in range(4):
            zbuf[r, pl.ds(q * 16, 16)] = jnp.zeros((16,), jnp.float32)
        return carry

    lax.fori_loop(0, K, _zb, 0)

    def _p1_outer(i, carry):
        pltpu.sync_copy(dst_hbm.at[pl.ds(slab + i * STG, STG)], dstg)

        def _p1_inner(j, c2):
            d = dstg[pl.ds(j * 16, 16)]
            c = lax.shift_right_logical(d, CSHIFT)
            plsc.addupdate_scatter(counts, [c * 16 + iota], ones)
            return c2

        lax.fori_loop(0, STG // 16, _p1_inner, 0)
        return carry

    lax.fori_loop(0, NSTG, _p1_outer, 0)

    def _pf(c, pstart):
        cntv = counts[pl.ds(c * 16, 16)]
        inc = plsc.cumsum(cntv)
        fill[pl.ds(c * 16, 16)] = pstart + (inc - cntv)
        pstarts[c] = pstart
        total = jnp.sum(cntv)
        return (pstart + total + (K - 1)) & (-K)

    pend = lax.fori_loop(0, NCHUNK, _pf, jnp.int32(0))
    pstarts[NCHUNK] = pend

    sent_d = jnp.full((16,), CHUNK, jnp.int32)

    def _sent(r, carry):
        for q in range(K // 16):
            sl = pl.ds(q * 16, 16)
            bk_sd[r, sl] = sent_d
            bk_wid[r, sl] = zeros16
        return carry

    lax.fori_loop(0, BROWS, _sent, 0)

    def _p2_outer(i, carry):
        base = slab + i * STG
        pltpu.sync_copy(dst_hbm.at[pl.ds(base, STG)], dstg)
        pltpu.sync_copy(src_hbm.at[pl.ds(base, STG)], srcg)

        def _p2_inner(j, c2):
            d = dstg[pl.ds(j * 16, 16)]
            s = srcg[pl.ds(j * 16, 16)]
            c = lax.shift_right_logical(d, CSHIFT)
            dloc = d & jnp.int32(CHUNK - 1)
            eid = base + j * 16 + iota
            wid = jnp.where(eid >= EW, eid - EW, eid)
            ci = c * 16 + iota
            pos = plsc.load_gather(fill, [ci])
            pr = lax.shift_right_logical(pos, 7)
            pc = pos & jnp.int32(K - 1)
            plsc.store_scatter(bk_sd, [pr, pc], s * jnp.int32(16384) + dloc)
            plsc.store_scatter(bk_wid, [pr, pc], wid)
            plsc.addupdate_scatter(fill, [ci], ones)
            return c2

        lax.fori_loop(0, STG // 16, _p2_inner, 0)
        return carry

    lax.fori_loop(0, NSTG, _p2_outer, 0)

    rows0 = sid * ROWS_PER_TILE
    for z in range(ROWS_PER_TILE // K):
        pltpu.sync_copy(zbuf, accsh.at[pl.ds(rows0 + z * K, K)])
    plsc.subcore_barrier()

    def _issue(row, xb, wb, idx, sx, sw):
        for q in range(K // 16):
            sl = pl.ds(q * 16, 16)
            idx[sl] = lax.shift_right_logical(bk_sd[row, sl], 14)
        pltpu.async_copy(x_hbm.at[idx], xb, sx)
        pltpu.async_copy(w_hbm.at[bk_wid.at[row]], wb, sw)

    def _drain(row, xb, wb, idx, sx, sw):
        pltpu.make_async_copy(x_hbm.at[idx], xb, sx).wait()
        pltpu.make_async_copy(w_hbm.at[bk_wid.at[row]], wb, sw).wait()

    def _proc(row, xb, wb):
        def _mul(r, c3):
            for q in range(4):
                sl = pl.ds(q * 16, 16)
                xb[r, sl] = xb[r, sl] * wb[r, sl]
            return c3

        lax.fori_loop(0, K, _mul, 0)
        for q in range(K // 16):
            sl = pl.ds(q * 16, 16)
            dtmp[sl] = bk_sd[row, sl] & jnp.int32(16383)
        pltpu.sync_copy(xb, accsh.at[dtmp], add=True)

    def _chunk(c, carry):
        p0 = pstarts[c]
        p1 = pstarts[c + 1]
        nblk = lax.shift_right_logical(p1 - p0, 7)
        r0 = lax.shift_right_logical(p0, 7)

        @pl.when(nblk > 0)
        def _():
            _issue(r0, xbuf, wbuf, idx0, semx, semw)

        def _pair(k, c2):
            b0 = 2 * k
            row0 = r0 + b0

            @pl.when(b0 + 1 < nblk)
            def _():
                _issue(row0 + 1, xbuf1, wbuf1, idx1, semx1, semw1)

            _drain(row0, xbuf, wbuf, idx0, semx, semw)
            _proc(row0, xbuf, wbuf)

            @pl.when(b0 + 2 < nblk)
            def _():
                _issue(row0 + 2, xbuf, wbuf, idx0, semx, semw)

            @pl.when(b0 + 1 < nblk)
            def _():
                _drain(row0 + 1, xbuf1, wbuf1, idx1, semx1, semw1)
                _proc(row0 + 1, xbuf1, wbuf1)

            return c2

        lax.fori_loop(0, (nblk + 1) >> 1, _pair, 0)
        plsc.subcore_barrier()

        ob = c * CHUNK + rows0
        pltpu.sync_copy(accsh.at[pl.ds(rows0, ROWS_PER_TILE)],
                        out_hbm.at[cid, pl.ds(ob, ROWS_PER_TILE)])
        for z in range(ROWS_PER_TILE // K):
            pltpu.sync_copy(zbuf, accsh.at[pl.ds(rows0 + z * K, K)])
        plsc.subcore_barrier()
        return carry

    lax.fori_loop(0, NCHUNK, _chunk, 0)


def _sc_scatter(dst, src, x, wcat):
    mesh = plsc.VectorSubcoreMesh(core_axis_name="c", subcore_axis_name="s")
    f = pl.kernel(
        _sc_body,
        mesh=mesh,
        compiler_params=pltpu.CompilerParams(
            needs_layout_passes=False, use_tc_tiling_on_sc=False),
        out_type=jax.ShapeDtypeStruct((2, NPAD, D), jnp.float32),
        scratch_types=[
            pltpu.VMEM((STG,), jnp.int32),
            pltpu.VMEM((STG,), jnp.int32),
            pltpu.VMEM((256,), jnp.int32),
            pltpu.VMEM((256,), jnp.int32),
            pltpu.VMEM((BROWS, K), jnp.int32),
            pltpu.VMEM((BROWS, K), jnp.int32),
            pltpu.VMEM((K,), jnp.int32),
            pltpu.VMEM((K,), jnp.int32),
            pltpu.VMEM((K,), jnp.int32),
            pltpu.VMEM((K, D), jnp.float32),
            pltpu.VMEM((K, D), jnp.float32),
            pltpu.VMEM((K, D), jnp.float32),
            pltpu.VMEM((K, D), jnp.float32),
            pltpu.VMEM((K, D), jnp.float32),
            pltpu.VMEM_SHARED((CHUNK + K, D), jnp.float32),
            pltpu.SMEM((NCHUNK + 1,), jnp.int32),
            pltpu.SemaphoreType.DMA,
            pltpu.SemaphoreType.DMA,
            pltpu.SemaphoreType.DMA,
            pltpu.SemaphoreType.DMA,
        ],
    )
    return f(dst, src, x, wcat)


def _final_body(pa_ref, pb_ref, a_ref, b_ref, w2_ref, b2_ref, w3_ref, b3_ref,
                o1_ref, o2_ref):
    i = pl.program_id(0)
    nb = pl.num_programs(0) // 2
    s = pa_ref[0] + pb_ref[0]
    h = jnp.dot(s, w2_ref[...], preferred_element_type=jnp.float32)
    h = _ss(h + b2_ref[...])
    y = jnp.dot(h, w3_ref[...], preferred_element_type=jnp.float32)
    y = y + b3_ref[...]

    @pl.when(i < nb)
    def _():
        o1_ref[...] = y + a_ref[...]

    @pl.when(i >= nb)
    def _():
        o2_ref[...] = y + b_ref[...]


def _final(partial, nf, nfg, W2, b2, W3, b3):
    blk = 1000
    nb = N1 // blk
    return pl.pallas_call(
        _final_body,
        grid=(2 * nb,),
        in_specs=[
            pl.BlockSpec((1, blk, D), lambda i: (0, i, 0)),
            pl.BlockSpec((1, blk, D), lambda i: (1, i, 0)),
            pl.BlockSpec((blk, D), lambda i: (jnp.minimum(i, nb - 1), 0)),
            pl.BlockSpec((blk, D), lambda i: (jnp.maximum(i - nb, 0), 0)),
            pl.BlockSpec((D, D), lambda i: (0, 0)),
            pl.BlockSpec((1, D), lambda i: (0, 0)),
            pl.BlockSpec((D, D), lambda i: (0, 0)),
            pl.BlockSpec((1, D), lambda i: (0, 0)),
        ],
        out_specs=[
            pl.BlockSpec((blk, D), lambda i: (jnp.minimum(i, nb - 1), 0)),
            pl.BlockSpec((blk, D), lambda i: (jnp.maximum(i - nb, 0), 0)),
        ],
        out_shape=[
            jax.ShapeDtypeStruct((N1, D), jnp.float32),
            jax.ShapeDtypeStruct((NG, D), jnp.float32),
        ],
    )(partial, partial, nf, nfg, W2, b2.reshape(1, D), W3, b3.reshape(1, D))


def kernel(edge_index1, edge_index2, node_feature, node_feature_ghost,
           rbf_tensor1, dist1, rbf_tensor2, dist2, cutoff1, cutoff2, exponent,
           dist2_0, dist2_1, W1, b1, W2, b2, W3, b3, fgW1, fgb1, fgW2, fgb2):
    x = _lin1(node_feature, node_feature_ghost, W1, b1)

    dshape = (E1 // 128, 128)
    env1d = 1.0 + jnp.cos(PI * dist1.reshape(dshape) / cutoff1)
    env1 = lax.optimization_barrier(env1d).reshape(E1, 1)
    ef = jnp.asarray(exponent, jnp.float32)

    def _poly(r):
        rpe = r ** ef
        return 1.0 + rpe * (ef * r - (ef + 1.0))

    env2d = (_poly(dist2.reshape(dshape) / cutoff2)
             * _poly(dist2_0.reshape(dshape) / cutoff1)
             * _poly(dist2_1.reshape(dshape) / cutoff1))
    env2 = lax.optimization_barrier(env2d).reshape(E2, 1)

    wcat = _filters(rbf_tensor1.T, rbf_tensor2.T, env1, env2, fgW1.T, fgb1,
                    fgW2, fgb2)

    dst = jnp.concatenate([edge_index1[0], edge_index2[2],
                           edge_index1[2], edge_index2[3]])
    src = jnp.concatenate([edge_index1[2], edge_index2[3],
                           edge_index1[0], edge_index2[2]])

    partial = _sc_scatter(dst, src, x, wcat)

    return _final(partial, node_feature, node_feature_ghost, W2, b2, W3, b3)
